# baseline (speedup 1.0000x reference)
import functools

import jax
import jax.numpy as jnp
from jax import lax
from jax.experimental import pallas as pl
from jax.experimental.pallas import tpu as pltpu
from jax.experimental.pallas import tpu_sc as plsc

D_MODEL = 64
LANES = 16
NUM_WORKERS = 32
TILE = 128
S_CHUNK = 5
T_UNROLL = 4

_MAGIC = 2.0 ** 112


def _make_kernel(n_tokens, seq):
    d_tiles = D_MODEL // 8
    n_tiles = n_tokens // TILE
    tiles_per_worker = n_tiles // NUM_WORKERS
    s_units = seq // S_CHUNK
    n_units = tiles_per_worker * s_units
    assert n_units % 2 == 0
    unit_rows = S_CHUNK * TILE
    idx_per_worker = tiles_per_worker * seq * TILE
    mesh = plsc.VectorSubcoreMesh(core_axis_name="c", subcore_axis_name="s")

    @functools.partial(
        pl.kernel,
        out_type=jax.ShapeDtypeStruct(
            (seq * d_tiles, n_tiles, 8, TILE), jnp.float32
        ),
        mesh=mesh,
        compiler_params=pltpu.CompilerParams(
            needs_layout_passes=False, use_tc_tiling_on_sc=False
        ),
        scratch_types=[
            pltpu.VMEM((seq * TILE,), jnp.int32),
            pltpu.VMEM((2, unit_rows, D_MODEL), jnp.float16),
            pltpu.VMEM((2, S_CHUNK * d_tiles, 8, TILE), jnp.float32),
            pltpu.SemaphoreType.DMA,
            pltpu.SemaphoreType.DMA,
            pltpu.SemaphoreType.DMA,
            pltpu.SemaphoreType.DMA,
        ],
    )
    def emb_kernel(idx_hbm, table_hbm, out_hbm, idx_v, in_v, out_v,
                   gsem0, gsem1, osem0, osem1):
        wid = lax.axis_index("s") * 2 + lax.axis_index("c")
        lane = lax.iota(jnp.int32, LANES)
        zeros = jnp.zeros((LANES,), jnp.int32)
        gsems = (gsem0, gsem1)
        osems = (osem0, osem1)

        grp = []
        for k in range(2):
            i_t = 4 * k + lax.shift_right_logical(lane, 2)
            r_lo = (2 * lane) & 7
            grp.append((i_t, r_lo, r_lo + 1))

        def unit_off(u):
            j4 = u // s_units
            s0 = (u % s_units) * S_CHUNK
            return j4, s0

        def stage_tile(j4):
            pltpu.sync_copy(
                idx_hbm.at[
                    pl.ds(
                        (wid * tiles_per_worker + j4) * (seq * TILE),
                        seq * TILE,
                    )
                ],
                idx_v,
            )

        def gather_slice(u):
            return idx_v.at[pl.ds((u % s_units) * unit_rows, unit_rows)]

        def start_gather(u, b):
            pltpu.async_copy(table_hbm.at[gather_slice(u)], in_v.at[b], gsems[b])

        def wait_gather(u, b):
            pltpu.make_async_copy(
                table_hbm.at[gather_slice(u)], in_v.at[b], gsems[b]
            ).wait()

        def wb_dst(u):
            j4, s0 = unit_off(u)
            jg = wid * tiles_per_worker + j4
            return out_hbm.at[pl.ds(s0 * d_tiles, S_CHUNK * d_tiles), jg]

        def start_wb(u, b):
            pltpu.async_copy(out_v.at[b], wb_dst(u), osems[b])

        def wait_wb(u, b):
            pltpu.make_async_copy(out_v.at[b], wb_dst(u), osems[b]).wait()

        def convert(b):
            def s_body(si, carry):
                sc = []
                for k in range(2):
                    i_t, r_lo, r_hi = grp[k]
                    sc.append((i_t + si * d_tiles, r_lo, r_hi))

                def t_body(t4, tc):
                    for dt in range(T_UNROLL):
                        t = t4 * T_UNROLL + dt
                        t_v = zeros + t
                        e = si * TILE + t
                        for k in range(2):
                            i_c, r_lo, r_hi = sc[k]
                            w16 = in_v[b, e, pl.ds(k * 2 * LANES, 2 * LANES)]
                            w = plsc.bitcast(w16, jnp.int32)
                            o_lo = ((w & 0x8000) << 16) | ((w & 0x7FFF) << 13)
                            o_hi = (w & -0x80000000) | (
                                lax.shift_right_logical(w, 3) & 0x0FFFE000
                            )
                            f_lo = lax.bitcast_convert_type(o_lo, jnp.float32) * _MAGIC
                            f_hi = lax.bitcast_convert_type(o_hi, jnp.float32) * _MAGIC
                            plsc.store_scatter(out_v.at[b], [i_c, r_lo, t_v], f_lo)
                            plsc.store_scatter(out_v.at[b], [i_c, r_hi, t_v], f_hi)
                    return tc

                lax.fori_loop(0, TILE // T_UNROLL, t_body, 0)
                return carry

            lax.fori_loop(0, S_CHUNK, s_body, 0)

        stage_tile(0)
        start_gather(0, 0)

        def outer(u2, carry):
            for b in range(2):
                u = 2 * u2 + b
                wait_gather(u, b)

                @pl.when(u + 1 < n_units)
                def _():
                    @pl.when((u + 1) % s_units == 0)
                    def _():
                        stage_tile((u + 1) // s_units)

                    start_gather(u + 1, 1 - b)

                @pl.when(u >= 2)
                def _():
                    wait_wb(u - 2, b)

                convert(b)
                start_wb(u, b)
            return carry

        lax.fori_loop(0, n_units // 2, outer, 0)

        for b in range(2):
            wait_wb(n_units - 2 + b, b)

    return emb_kernel


def kernel(token_ids, weight):
    n_tokens, seq = token_ids.shape
    n_tiles = n_tokens // TILE
    idx = token_ids.astype(jnp.int32).T
    idx = idx.reshape(seq, n_tiles, TILE).transpose(1, 0, 2).reshape(-1)
    phys = _make_kernel(n_tokens, seq)(idx, weight)
    phys = phys.reshape(seq, D_MODEL // 8, n_tiles, 8, TILE)
    return phys.transpose(2, 4, 0, 1, 3).reshape(n_tokens, seq, D_MODEL)


# speedup vs baseline: 1.1579x; 1.1579x over previous
import functools

import jax
import jax.numpy as jnp
from jax import lax
from jax.experimental import pallas as pl
from jax.experimental.pallas import tpu as pltpu
from jax.experimental.pallas import tpu_sc as plsc

D_MODEL = 64
LANES = 16
NUM_WORKERS = 32
CHUNK_TROWS = 8
TOK_UNROLL = 5

_MAGIC = 2.0 ** 112


def _f16_bits_to_f32(h):
    o = ((h & 0x8000) << 16) | ((h & 0x7FFF) << 13)
    return lax.bitcast_convert_type(o, jnp.float32) * _MAGIC


def _make_kernel(n_tokens, seq):
    trows_per_worker = n_tokens // NUM_WORKERS
    n_chunks = trows_per_worker // CHUNK_TROWS
    assert n_chunks % 2 == 0
    chunk_rows = CHUNK_TROWS * seq
    mesh = plsc.VectorSubcoreMesh(core_axis_name="c", subcore_axis_name="s")

    @functools.partial(
        pl.kernel,
        out_type=jax.ShapeDtypeStruct((n_tokens, seq, D_MODEL), jnp.float32),
        mesh=mesh,
        compiler_params=pltpu.CompilerParams(
            needs_layout_passes=False, use_tc_tiling_on_sc=False
        ),
        scratch_types=[
            pltpu.VMEM((trows_per_worker * seq,), jnp.int32),
            pltpu.VMEM((2, chunk_rows, D_MODEL), jnp.float16),
            pltpu.VMEM((2, CHUNK_TROWS, seq, D_MODEL), jnp.float32),
            pltpu.SemaphoreType.DMA,
            pltpu.SemaphoreType.DMA,
            pltpu.SemaphoreType.DMA,
            pltpu.SemaphoreType.DMA,
        ],
    )
    def emb_kernel(idx_hbm, table_hbm, out_hbm, idx_v, in_v, out_v,
                   gsem0, gsem1, osem0, osem1):
        wid = lax.axis_index("s") * 2 + lax.axis_index("c")
        w_row_base = wid * trows_per_worker
        lane = lax.iota(jnp.int32, LANES)
        zeros = jnp.zeros((LANES,), jnp.int32)
        cols = []
        for k in range(2):
            cols.append((2 * lane + 2 * LANES * k, 2 * lane + 2 * LANES * k + 1))
        gsems = (gsem0, gsem1)
        osems = (osem0, osem1)

        def start_gather(c, b):
            idx_slice = idx_v.at[pl.ds(c * chunk_rows, chunk_rows)]
            pltpu.async_copy(table_hbm.at[idx_slice], in_v.at[b], gsems[b])

        def wait_gather(c, b):
            idx_slice = idx_v.at[pl.ds(c * chunk_rows, chunk_rows)]
            pltpu.make_async_copy(
                table_hbm.at[idx_slice], in_v.at[b], gsems[b]
            ).wait()

        def convert(b):
            def trow_body(tr, carry):
                trs = zeros + tr

                def tok_body(t2, tcarry):
                    for dt in range(TOK_UNROLL):
                        t = t2 * TOK_UNROLL + dt
                        ts = zeros + t
                        e = tr * seq + t
                        for k, (ce, co) in enumerate(cols):
                            w16 = in_v[b, e, pl.ds(k * 2 * LANES, 2 * LANES)]
                            w = plsc.bitcast(w16, jnp.int32)
                            o_lo = ((w & 0x8000) << 16) | ((w & 0x7FFF) << 13)
                            o_hi = (w & -0x80000000) | (
                                lax.shift_right_logical(w, 3) & 0x0FFFE000
                            )
                            f_lo = lax.bitcast_convert_type(o_lo, jnp.float32) * _MAGIC
                            f_hi = lax.bitcast_convert_type(o_hi, jnp.float32) * _MAGIC
                            plsc.store_scatter(out_v.at[b], [trs, ts, ce], f_lo)
                            plsc.store_scatter(out_v.at[b], [trs, ts, co], f_hi)
                    return tcarry

                lax.fori_loop(0, seq // TOK_UNROLL, tok_body, 0)
                return carry

            lax.fori_loop(0, CHUNK_TROWS, trow_body, 0)

        pltpu.sync_copy(
            idx_hbm.at[pl.ds(w_row_base * seq, trows_per_worker * seq)], idx_v
        )
        start_gather(0, 0)

        def outer(c2, carry):
            for b in range(2):
                c = 2 * c2 + b
                @pl.when(c + 1 < n_chunks)
                def _():
                    start_gather(c + 1, 1 - b)

                wait_gather(c, b)
                @pl.when(c >= 2)
                def _():
                    pltpu.make_async_copy(
                        out_v.at[b],
                        out_hbm.at[pl.ds(w_row_base, CHUNK_TROWS)],
                        osems[b],
                    ).wait()

                convert(b)
                row_base = w_row_base + c * CHUNK_TROWS
                pltpu.async_copy(
                    out_v.at[b], out_hbm.at[pl.ds(row_base, CHUNK_TROWS)], osems[b]
                )
            return carry

        lax.fori_loop(0, n_chunks // 2, outer, 0)

        for b in range(2):
            pltpu.make_async_copy(
                out_v.at[b], out_hbm.at[pl.ds(w_row_base, CHUNK_TROWS)], osems[b]
            ).wait()

    return emb_kernel


def kernel(token_ids, weight):
    n_tokens, seq = token_ids.shape
    idx = token_ids.reshape(-1).astype(jnp.int32)
    out = _make_kernel(n_tokens, seq)(idx, weight)
    return out


# speedup vs baseline: 1.1589x; 1.0009x over previous
import functools

import jax
import jax.numpy as jnp
from jax import lax
from jax.experimental import pallas as pl
from jax.experimental.pallas import tpu as pltpu
from jax.experimental.pallas import tpu_sc as plsc

D_MODEL = 64
LANES = 16
NUM_WORKERS = 32
CHUNK_TROWS = 8
TOK_UNROLL = 10

_MAGIC = 2.0 ** 112


def _f16_bits_to_f32(h):
    o = ((h & 0x8000) << 16) | ((h & 0x7FFF) << 13)
    return lax.bitcast_convert_type(o, jnp.float32) * _MAGIC


def _make_kernel(n_tokens, seq):
    trows_per_worker = n_tokens // NUM_WORKERS
    n_chunks = trows_per_worker // CHUNK_TROWS
    assert n_chunks % 2 == 0
    chunk_rows = CHUNK_TROWS * seq
    mesh = plsc.VectorSubcoreMesh(core_axis_name="c", subcore_axis_name="s")

    @functools.partial(
        pl.kernel,
        out_type=jax.ShapeDtypeStruct((n_tokens, seq, D_MODEL), jnp.float32),
        mesh=mesh,
        compiler_params=pltpu.CompilerParams(
            needs_layout_passes=False, use_tc_tiling_on_sc=False
        ),
        scratch_types=[
            pltpu.VMEM((trows_per_worker * seq,), jnp.int32),
            pltpu.VMEM((2, chunk_rows, D_MODEL), jnp.float16),
            pltpu.VMEM((2, CHUNK_TROWS, seq, D_MODEL), jnp.float32),
            pltpu.SemaphoreType.DMA,
            pltpu.SemaphoreType.DMA,
            pltpu.SemaphoreType.DMA,
            pltpu.SemaphoreType.DMA,
        ],
    )
    def emb_kernel(idx_hbm, table_hbm, out_hbm, idx_v, in_v, out_v,
                   gsem0, gsem1, osem0, osem1):
        wid = lax.axis_index("s") * 2 + lax.axis_index("c")
        w_row_base = wid * trows_per_worker
        lane = lax.iota(jnp.int32, LANES)
        zeros = jnp.zeros((LANES,), jnp.int32)
        cols = []
        for k in range(2):
            cols.append((2 * lane + 2 * LANES * k, 2 * lane + 2 * LANES * k + 1))
        gsems = (gsem0, gsem1)
        osems = (osem0, osem1)

        def start_gather(c, b):
            idx_slice = idx_v.at[pl.ds(c * chunk_rows, chunk_rows)]
            pltpu.async_copy(table_hbm.at[idx_slice], in_v.at[b], gsems[b])

        def wait_gather(c, b):
            idx_slice = idx_v.at[pl.ds(c * chunk_rows, chunk_rows)]
            pltpu.make_async_copy(
                table_hbm.at[idx_slice], in_v.at[b], gsems[b]
            ).wait()

        def convert(b):
            def trow_body(tr, carry):
                trs = zeros + tr

                def tok_body(t2, tcarry):
                    for dt in range(TOK_UNROLL):
                        t = t2 * TOK_UNROLL + dt
                        ts = zeros + t
                        e = tr * seq + t
                        for k, (ce, co) in enumerate(cols):
                            w16 = in_v[b, e, pl.ds(k * 2 * LANES, 2 * LANES)]
                            w = plsc.bitcast(w16, jnp.int32)
                            o_lo = ((w & 0x8000) << 16) | ((w & 0x7FFF) << 13)
                            o_hi = (w & -0x80000000) | (
                                lax.shift_right_logical(w, 3) & 0x0FFFE000
                            )
                            f_lo = lax.bitcast_convert_type(o_lo, jnp.float32) * _MAGIC
                            f_hi = lax.bitcast_convert_type(o_hi, jnp.float32) * _MAGIC
                            plsc.store_scatter(out_v.at[b], [trs, ts, ce], f_lo)
                            plsc.store_scatter(out_v.at[b], [trs, ts, co], f_hi)
                    return tcarry

                lax.fori_loop(0, seq // TOK_UNROLL, tok_body, 0)
                return carry

            lax.fori_loop(0, CHUNK_TROWS, trow_body, 0)

        pltpu.sync_copy(
            idx_hbm.at[pl.ds(w_row_base * seq, trows_per_worker * seq)], idx_v
        )
        start_gather(0, 0)

        def outer(c2, carry):
            for b in range(2):
                c = 2 * c2 + b
                @pl.when(c + 1 < n_chunks)
                def _():
                    start_gather(c + 1, 1 - b)

                wait_gather(c, b)
                @pl.when(c >= 2)
                def _():
                    pltpu.make_async_copy(
                        out_v.at[b],
                        out_hbm.at[pl.ds(w_row_base, CHUNK_TROWS)],
                        osems[b],
                    ).wait()

                convert(b)
                row_base = w_row_base + c * CHUNK_TROWS
                pltpu.async_copy(
                    out_v.at[b], out_hbm.at[pl.ds(row_base, CHUNK_TROWS)], osems[b]
                )
            return carry

        lax.fori_loop(0, n_chunks // 2, outer, 0)

        for b in range(2):
            pltpu.make_async_copy(
                out_v.at[b], out_hbm.at[pl.ds(w_row_base, CHUNK_TROWS)], osems[b]
            ).wait()

    return emb_kernel


def kernel(token_ids, weight):
    n_tokens, seq = token_ids.shape
    idx = token_ids.reshape(-1).astype(jnp.int32)
    out = _make_kernel(n_tokens, seq)(idx, weight)
    return out
